# baseline (speedup 1.0000x reference)
import functools

import jax
import jax.numpy as jnp
from jax import lax
from jax.experimental import pallas as pl
from jax.experimental.pallas import tpu as pltpu
from jax.experimental.pallas import tpu_sc as plsc

_CHUNK = 128
_LANES = 16


def _pos_encoding(length: int, hidden: int) -> jax.Array:
    depth = hidden // 2
    positions = jnp.arange(length)[:, None].astype(jnp.float32)
    depths = jnp.arange(depth)[None, :].astype(jnp.float32) / depth
    angle_rates = 1.0 / (10000.0 ** depths)
    angle_rads = positions * angle_rates
    return jnp.concatenate(
        [jnp.sin(angle_rads), jnp.cos(angle_rads)], axis=-1
    ).astype(jnp.float32)


@functools.partial(jax.jit, static_argnames=("n_total", "hidden", "length"))
def _emb_lookup(x_flat, pos, table, *, n_total, hidden, length):
    info = plsc.get_sparse_core_info()
    nc, ns = info.num_cores, info.num_subcores
    nw = nc * ns
    per_w = n_total // nw
    n_chunks = per_w // _CHUNK
    scale = float(hidden) ** 0.5
    vregs_per_row = hidden // _LANES

    mesh = plsc.VectorSubcoreMesh(core_axis_name="c", subcore_axis_name="s")

    @functools.partial(
        pl.kernel,
        mesh=mesh,
        compiler_params=pltpu.CompilerParams(use_tc_tiling_on_sc=False),
        out_type=jax.ShapeDtypeStruct((n_total, hidden), jnp.float32),
        scratch_types=[
            pltpu.VMEM((_CHUNK,), jnp.int32),
            pltpu.VMEM((_CHUNK, hidden), jnp.float32),
            pltpu.VMEM((length, hidden), jnp.float32),
            pltpu.SemaphoreType.DMA,
        ],
    )
    def k(x_hbm, pos_hbm, table_hbm, out_hbm, idx_v, rows_v, pos_v, sem):
        wid = lax.axis_index("s") * nc + lax.axis_index("c")
        base = wid * per_w
        pltpu.sync_copy(pos_hbm, pos_v)

        def chunk_body(g, carry):
            row0 = base + g * _CHUNK
            pltpu.sync_copy(x_hbm.at[pl.ds(row0, _CHUNK)], idx_v)
            pltpu.async_copy(table_hbm.at[idx_v], rows_v, sem).wait()
            phase = lax.rem(g * _CHUNK, length)

            def row_body(r, carry2):
                p = phase + r
                p = jnp.where(p >= length, p - length, p)
                for c in range(vregs_per_row):
                    sl = pl.ds(c * _LANES, _LANES)
                    rows_v[r, sl] = rows_v[r, sl] * scale + pos_v[p, sl]
                return carry2

            lax.fori_loop(0, _CHUNK, row_body, 0, unroll=2)
            pltpu.sync_copy(rows_v, out_hbm.at[pl.ds(row0, _CHUNK)])
            return carry

        lax.fori_loop(0, n_chunks, chunk_body, 0)

    return k(x_flat, pos, table)


def kernel(x, table):
    b, length = x.shape
    hidden = table.shape[1]
    n_total = b * length
    pos = _pos_encoding(length, hidden)
    out = _emb_lookup(
        x.reshape(n_total), pos, table,
        n_total=n_total, hidden=hidden, length=length,
    )
    return out.reshape(b, length, hidden)


# speedup vs baseline: 1.1848x; 1.1848x over previous
import functools

import jax
import jax.numpy as jnp
from jax import lax
from jax.experimental import pallas as pl
from jax.experimental.pallas import tpu as pltpu
from jax.experimental.pallas import tpu_sc as plsc

_CHUNK = 128
_LANES = 16


def _pos_encoding(length: int, hidden: int) -> jax.Array:
    depth = hidden // 2
    positions = jnp.arange(length)[:, None].astype(jnp.float32)
    depths = jnp.arange(depth)[None, :].astype(jnp.float32) / depth
    angle_rates = 1.0 / (10000.0 ** depths)
    angle_rads = positions * angle_rates
    return jnp.concatenate(
        [jnp.sin(angle_rads), jnp.cos(angle_rads)], axis=-1
    ).astype(jnp.float32)


@functools.partial(jax.jit, static_argnames=("n_total", "hidden", "length"))
def _emb_lookup(x2d, pos, table, *, n_total, hidden, length):
    info = plsc.get_sparse_core_info()
    nc, ns = info.num_cores, info.num_subcores
    nw = nc * ns
    per_w = n_total // nw
    n_chunks = per_w // _CHUNK
    scale = float(hidden) ** 0.5
    vregs_per_row = hidden // _LANES

    mesh = plsc.VectorSubcoreMesh(core_axis_name="c", subcore_axis_name="s")

    @functools.partial(
        pl.kernel,
        mesh=mesh,
        compiler_params=pltpu.CompilerParams(use_tc_tiling_on_sc=False),
        out_type=jax.ShapeDtypeStruct((n_total, hidden), jnp.float32),
        scratch_types=[
            pltpu.VMEM((n_chunks, _CHUNK), jnp.int32),
            pltpu.VMEM((_CHUNK, hidden), jnp.float32),
            pltpu.VMEM((_CHUNK, hidden), jnp.float32),
            pltpu.VMEM((length, hidden), jnp.float32),
            pltpu.SemaphoreType.DMA,
            pltpu.SemaphoreType.DMA,
            pltpu.SemaphoreType.DMA,
            pltpu.SemaphoreType.DMA,
        ],
    )
    def k(x_hbm, pos_hbm, table_hbm, out_hbm, idx_all, rows0, rows1,
          pos_v, sg0, sg1, so0, so1):
        wid = lax.axis_index("s") * nc + lax.axis_index("c")
        base = wid * per_w
        rows = (rows0, rows1)
        sem_g = (sg0, sg1)
        sem_o = (so0, so1)

        pltpu.sync_copy(x_hbm.at[pl.ds(wid * n_chunks, n_chunks)], idx_all)
        pltpu.sync_copy(pos_hbm, pos_v)

        def gather_copy(g, b):
            return pltpu.make_async_copy(
                table_hbm.at[idx_all.at[g]], rows[b], sem_g[b])

        def store_copy(g, b):
            return pltpu.make_async_copy(
                rows[b], out_hbm.at[pl.ds(base + g * _CHUNK, _CHUNK)],
                sem_o[b])

        def step(g, b):
            @pl.when(g + 1 < n_chunks)
            def _():
                @pl.when(g >= 1)
                def _():
                    store_copy(g - 1, 1 - b).wait()
                gather_copy(g + 1, 1 - b).start()

            gather_copy(g, b).wait()
            phase = lax.rem(g * _CHUNK, length)
            rv = rows[b]

            def row_body(r, carry):
                p = phase + r
                p = jnp.where(p >= length, p - length, p)
                for c in range(vregs_per_row):
                    sl = pl.ds(c * _LANES, _LANES)
                    rv[r, sl] = rv[r, sl] * scale + pos_v[p, sl]
                return carry

            lax.fori_loop(0, _CHUNK, row_body, 0, unroll=4)
            store_copy(g, b).start()

        def pair(i, carry):
            step(2 * i + 0, 0)
            step(2 * i + 1, 1)
            return carry

        gather_copy(0, 0).start()
        lax.fori_loop(0, n_chunks // 2, pair, 0)
        store_copy(n_chunks - 2, 0).wait()
        store_copy(n_chunks - 1, 1).wait()

    return k(x2d, pos, table)


def kernel(x, table):
    b, length = x.shape
    hidden = table.shape[1]
    n_total = b * length
    pos = _pos_encoding(length, hidden)
    out = _emb_lookup(
        x.reshape(n_total // _CHUNK, _CHUNK), pos, table,
        n_total=n_total, hidden=hidden, length=length,
    )
    return out.reshape(b, length, hidden)
